# baseline (speedup 1.0000x reference)
import functools

import jax
import jax.numpy as jnp
from jax import lax
from jax.experimental import pallas as pl
from jax.experimental.pallas import tpu as pltpu
from jax.experimental.pallas import tpu_sc as plsc

B = 4096
D = 64
L = 16
NC = 2
NS = 16
NW = NC * NS
BPW = B // NW


def _sc_diff_body(uf_hbm, if_hbm, uidx_hbm, pidx_hbm, nidx_hbm, out_hbm,
                  uidx_v, pidx_v, nidx_v, urows_v, prows_v, nrows_v,
                  diffs_v, sem_u, sem_p, sem_n):
    wid = lax.axis_index("s") * NC + lax.axis_index("c")
    base = wid * BPW
    pltpu.sync_copy(uidx_hbm.at[pl.ds(base, BPW)], uidx_v)
    pltpu.sync_copy(pidx_hbm.at[pl.ds(base, BPW)], pidx_v)
    pltpu.sync_copy(nidx_hbm.at[pl.ds(base, BPW)], nidx_v)
    cu = pltpu.async_copy(uf_hbm.at[uidx_v], urows_v, sem_u)
    cp = pltpu.async_copy(if_hbm.at[pidx_v], prows_v, sem_p)
    cn = pltpu.async_copy(if_hbm.at[nidx_v], nrows_v, sem_n)
    cu.wait()
    cp.wait()
    cn.wait()

    iota = lax.iota(jnp.int32, L)

    def group(g, carry):
        rows = g * L + iota
        acc = jnp.zeros((L,), jnp.float32)
        for k in range(D):
            col = jnp.full((L,), k, jnp.int32)
            u = plsc.load_gather(urows_v, [rows, col])
            p = plsc.load_gather(prows_v, [rows, col])
            n = plsc.load_gather(nrows_v, [rows, col])
            acc = acc + u * (n - p)
        diffs_v[pl.ds(g * L, L)] = acc
        return carry

    lax.fori_loop(0, BPW // L, group, 0)
    pltpu.sync_copy(diffs_v, out_hbm.at[pl.ds(base, BPW)])


@jax.jit
def _sc_diff(users_feature, items_feature, uidx, pidx, nidx):
    mesh = plsc.VectorSubcoreMesh(core_axis_name="c", subcore_axis_name="s")
    return pl.kernel(
        _sc_diff_body,
        out_type=jax.ShapeDtypeStruct((B,), jnp.float32),
        mesh=mesh,
        scratch_types=[
            pltpu.VMEM((BPW,), jnp.int32),
            pltpu.VMEM((BPW,), jnp.int32),
            pltpu.VMEM((BPW,), jnp.int32),
            pltpu.VMEM((BPW, D), jnp.float32),
            pltpu.VMEM((BPW, D), jnp.float32),
            pltpu.VMEM((BPW, D), jnp.float32),
            pltpu.VMEM((BPW,), jnp.float32),
            pltpu.SemaphoreType.DMA,
            pltpu.SemaphoreType.DMA,
            pltpu.SemaphoreType.DMA,
        ],
        compiler_params=pltpu.CompilerParams(
            needs_layout_passes=False, use_tc_tiling_on_sc=False),
    )(users_feature, items_feature, uidx, pidx, nidx)


def _tc_loss_body(x_ref, o_ref):
    x = x_ref[...]
    sp = jnp.maximum(x, 0.0) + jnp.log1p(jnp.exp(-jnp.abs(x)))
    o_ref[0, 0] = jnp.sum(sp) * (1.0 / B)


@jax.jit
def _tc_loss(diffs2d):
    out = pl.pallas_call(
        _tc_loss_body,
        out_shape=jax.ShapeDtypeStruct((1, 1), jnp.float32),
        out_specs=pl.BlockSpec(memory_space=pltpu.SMEM),
    )(diffs2d)
    return out[0, 0]


def kernel(users, items, users_feature, items_feature):
    uidx = users.reshape(B)
    pidx = items[:, 0]
    nidx = items[:, 1]
    diffs = _sc_diff(users_feature, items_feature, uidx, pidx, nidx)
    return _tc_loss(diffs.reshape(NW, BPW))
